# Initial kernel scaffold; baseline (speedup 1.0000x reference)
import jax
import jax.numpy as jnp
from jax.experimental import pallas as pl


def kernel(x, table, W, b):
    raise NotImplementedError("write your pallas kernel here")


# speedup vs baseline: 12.1976x; 12.1976x over previous
import jax
import jax.numpy as jnp
from jax import lax
from jax.experimental import pallas as pl
from jax.experimental.pallas import tpu as pltpu
from jax.experimental.pallas import tpu_sc as plsc

_INFO = plsc.get_sparse_core_info()
_NC = _INFO.num_cores
_NS = _INFO.num_subcores
_NW = _NC * _NS

_GRP = 128
_FIRE = 8


def _gather_body(table_hbm, idx_hbm, out_hbm, idx_v, rows_v, sem):
    wid = lax.axis_index("s") * _NC + lax.axis_index("c")
    G = idx_hbm.shape[1]
    rows_per_outer = _FIRE * _GRP
    pltpu.sync_copy(idx_hbm.at[wid], idx_v)

    def outer(o, carry):
        handles = []
        for k in range(_FIRE):
            g = o * _FIRE + k
            handles.append(
                pltpu.async_copy(
                    table_hbm.at[idx_v.at[g]],
                    rows_v.at[pl.ds(k * _GRP, _GRP)],
                    sem,
                )
            )
        for h in handles:
            h.wait()
        off = pl.multiple_of(
            wid * (G * _GRP) + o * rows_per_outer, rows_per_outer
        )
        pltpu.sync_copy(rows_v, out_hbm.at[pl.ds(off, rows_per_outer)])
        return carry

    lax.fori_loop(0, G // _FIRE, outer, 0)


def _sc_gather(table, idx3, n_rows):
    D = table.shape[1]
    G = idx3.shape[1]
    call = pl.kernel(
        _gather_body,
        out_type=jax.ShapeDtypeStruct((n_rows, D), jnp.float32),
        mesh=plsc.VectorSubcoreMesh(core_axis_name="c", subcore_axis_name="s"),
        scratch_types=[
            pltpu.VMEM((G, _GRP), jnp.int32),
            pltpu.VMEM((_FIRE * _GRP, D), jnp.float32),
            pltpu.SemaphoreType.DMA,
        ],
        compiler_params=pltpu.CompilerParams(use_tc_tiling_on_sc=False),
    )
    return call(table, idx3)


def _proj_body(emb_ref, w_ref, b_ref, out_ref):
    out_ref[...] = (
        jnp.dot(emb_ref[...], w_ref[...], preferred_element_type=jnp.float32)
        + b_ref[...]
    )


def _tc_project(emb, W, b2):
    N, D = emb.shape
    Dout = W.shape[1]
    BR = 8192
    return pl.pallas_call(
        _proj_body,
        grid=(N // BR,),
        in_specs=[
            pl.BlockSpec((BR, D), lambda i: (i, 0)),
            pl.BlockSpec((D, Dout), lambda i: (0, 0)),
            pl.BlockSpec((1, Dout), lambda i: (0, 0)),
        ],
        out_specs=pl.BlockSpec((BR, Dout), lambda i: (i, 0)),
        out_shape=jax.ShapeDtypeStruct((N, Dout), jnp.float32),
    )(emb, W, b2)


def kernel(x, table, W, b):
    B, L = x.shape
    Dout = W.shape[1]
    N = B * L
    idx3 = x.reshape(_NW, N // (_NW * _GRP), _GRP).astype(jnp.int32)
    emb = _sc_gather(table, idx3, N)
    out = _tc_project(emb, W, b.reshape(1, Dout))
    return out.reshape(B, L, Dout)


# speedup vs baseline: 14.3536x; 1.1768x over previous
import jax
import jax.numpy as jnp
from jax import lax
from jax.experimental import pallas as pl
from jax.experimental.pallas import tpu as pltpu
from jax.experimental.pallas import tpu_sc as plsc

_INFO = plsc.get_sparse_core_info()
_NC = _INFO.num_cores
_NS = _INFO.num_subcores
_NW = _NC * _NS

_GRP = 128
_FIRE = 8


def _gather_body(table_hbm, idx_hbm, out_hbm, idx_v, rows_v, sem):
    wid = lax.axis_index("s") * _NC + lax.axis_index("c")
    G = idx_hbm.shape[1]
    rows_per_outer = _FIRE * _GRP
    pltpu.sync_copy(idx_hbm.at[wid], idx_v)

    def outer(o, carry):
        handles = []
        for k in range(_FIRE):
            g = o * _FIRE + k
            handles.append(
                pltpu.async_copy(
                    table_hbm.at[idx_v.at[g]],
                    rows_v.at[pl.ds(k * _GRP, _GRP)],
                    sem,
                )
            )
        for h in handles:
            h.wait()
        off = pl.multiple_of(
            wid * (G * _GRP) + o * rows_per_outer, rows_per_outer
        )
        pltpu.sync_copy(rows_v, out_hbm.at[pl.ds(off, rows_per_outer)])
        return carry

    lax.fori_loop(0, G // _FIRE, outer, 0)


def _sc_gather(table, idx3, n_rows):
    D = table.shape[1]
    G = idx3.shape[1]
    call = pl.kernel(
        _gather_body,
        out_type=jax.ShapeDtypeStruct((n_rows, D), jnp.float32),
        mesh=plsc.VectorSubcoreMesh(core_axis_name="c", subcore_axis_name="s"),
        scratch_types=[
            pltpu.VMEM((G, _GRP), jnp.int32),
            pltpu.VMEM((_FIRE * _GRP, D), jnp.float32),
            pltpu.SemaphoreType.DMA,
        ],
        compiler_params=pltpu.CompilerParams(use_tc_tiling_on_sc=False),
    )
    return call(table, idx3)


def _proj_body(emb_ref, w_ref, b_ref, out_ref):
    e = emb_ref[...]
    w = w_ref[...]
    ys = [
        jnp.dot(e[:, 32 * q : 32 * (q + 1)], w, preferred_element_type=jnp.float32)
        for q in range(4)
    ]
    y = jnp.stack(ys, axis=1)
    bb, ll, dd = out_ref.shape
    out_ref[...] = y.reshape(bb, ll, dd) + b_ref[...]


def _tc_project(emb2, W, b2, B, L):
    Dout = W.shape[1]
    D = W.shape[0]
    BB = 128
    M4 = BB * L // 4
    return pl.pallas_call(
        _proj_body,
        grid=(B // BB,),
        in_specs=[
            pl.BlockSpec((M4, 128), lambda i: (i, 0)),
            pl.BlockSpec((D, Dout), lambda i: (0, 0)),
            pl.BlockSpec((1, Dout), lambda i: (0, 0)),
        ],
        out_specs=pl.BlockSpec((BB, L, Dout), lambda i: (i, 0, 0)),
        out_shape=jax.ShapeDtypeStruct((B, L, Dout), jnp.float32),
    )(emb2, W, b2)


def kernel(x, table, W, b):
    B, L = x.shape
    Dout = W.shape[1]
    N = B * L
    idx3 = x.reshape(_NW, N // (_NW * _GRP), _GRP).astype(jnp.int32)
    emb = _sc_gather(table, idx3, N)
    emb2 = emb.reshape(N // 4, 128)
    return _tc_project(emb2, W, b.reshape(1, Dout), B, L)


# speedup vs baseline: 25.4184x; 1.7709x over previous
import jax
import jax.numpy as jnp
from jax import lax
from jax.experimental import pallas as pl
from jax.experimental.pallas import tpu as pltpu
from jax.experimental.pallas import tpu_sc as plsc

_INFO = plsc.get_sparse_core_info()
_NC = _INFO.num_cores
_NS = _INFO.num_subcores
_NW = _NC * _NS

_GRP = 128
_FIRE = 8


def _gather_body(table_hbm, idx_hbm, out_hbm, idx_v, rows_v, sem):
    wid = lax.axis_index("s") * _NC + lax.axis_index("c")
    G = idx_hbm.shape[1]
    rows_per_outer = _FIRE * _GRP
    pltpu.sync_copy(idx_hbm.at[wid], idx_v)

    def outer(o, carry):
        handles = []
        for k in range(_FIRE):
            g = o * _FIRE + k
            handles.append(
                pltpu.async_copy(
                    table_hbm.at[idx_v.at[g]],
                    rows_v.at[pl.ds(k * _GRP, _GRP)],
                    sem,
                )
            )
        for h in handles:
            h.wait()
        off = pl.multiple_of(
            wid * (G * _GRP) + o * rows_per_outer, rows_per_outer
        )
        pltpu.sync_copy(rows_v, out_hbm.at[pl.ds(off, rows_per_outer)])
        return carry

    lax.fori_loop(0, G // _FIRE, outer, 0)


def _sc_gather(table, idx3, n_rows):
    D = table.shape[1]
    G = idx3.shape[1]
    call = pl.kernel(
        _gather_body,
        out_type=jax.ShapeDtypeStruct((n_rows, D), jnp.float32),
        mesh=plsc.VectorSubcoreMesh(core_axis_name="c", subcore_axis_name="s"),
        scratch_types=[
            pltpu.VMEM((G, _GRP), jnp.int32),
            pltpu.VMEM((_FIRE * _GRP, D), jnp.float32),
            pltpu.SemaphoreType.DMA,
        ],
        compiler_params=pltpu.CompilerParams(use_tc_tiling_on_sc=False),
    )
    return call(table, idx3)


_BB = 4096


def _proj_body(e_ref, wt_ref, b_ref, out_ref):
    wt = wt_ref[...]
    bcol = b_ref[...]
    e = e_ref[...]
    parts = [
        jax.lax.dot_general(
            wt,
            e[:, 32 * q : 32 * (q + 1)],
            (((1,), (1,)), ((), ())),
            preferred_element_type=jnp.float32,
        )
        for q in range(4)
    ]
    yt = jnp.concatenate(parts, axis=1)
    out_ref[...] = (yt + bcol)[None]


def _tc_project(emb2, Wt, b2, B, L):
    Dout, D = Wt.shape
    nI = B // _BB
    M = _BB // 4
    return pl.pallas_call(
        _proj_body,
        grid=(L, nI),
        in_specs=[
            pl.BlockSpec((M, 128), lambda l, i: (l * nI + i, 0)),
            pl.BlockSpec((Dout, D), lambda l, i: (0, 0)),
            pl.BlockSpec((Dout, 1), lambda l, i: (0, 0)),
        ],
        out_specs=pl.BlockSpec((1, Dout, _BB), lambda l, i: (l, 0, i)),
        out_shape=jax.ShapeDtypeStruct((L, Dout, B), jnp.float32),
    )(emb2, Wt, b2)


def kernel(x, table, W, b):
    B, L = x.shape
    Dout = W.shape[1]
    N = B * L
    Q = _BB // 4
    xT = jnp.transpose(x).astype(jnp.int32)
    idxp = xT.reshape(N // _BB, 4, Q).transpose(0, 2, 1).reshape(N)
    idx3 = idxp.reshape(_NW, N // (_NW * _GRP), _GRP)
    emb = _sc_gather(table, idx3, N)
    emb2 = emb.reshape(N // 4, 128)
    outT = _tc_project(emb2, jnp.transpose(W), b.reshape(Dout, 1), B, L)
    return jnp.transpose(outT, (2, 0, 1))
